# Initial kernel scaffold; baseline (speedup 1.0000x reference)
import jax
import jax.numpy as jnp
from jax.experimental import pallas as pl


def kernel(inputs, token_table, pos_table):
    raise NotImplementedError("write your pallas kernel here")


# speedup vs baseline: 3.1036x; 3.1036x over previous
import functools

import jax
import jax.numpy as jnp
from jax import lax
from jax.experimental import pallas as pl
from jax.experimental.pallas import tpu as pltpu
from jax.experimental.pallas import tpu_sc as plsc

SEQ = 200
DIM = 64
NUM_CORES = 2
NUM_SUBCORES = 16
NUM_WORKERS = NUM_CORES * NUM_SUBCORES
GATHER_CHUNK = 40


def _body(table_hbm, idx_hbm, pos_hbm, out_hbm, idx_v, buf, pos_v, sem):
    n_seq = idx_hbm.shape[0] // SEQ
    seq_per_w = n_seq // NUM_WORKERS
    wid = lax.axis_index("s") * NUM_CORES + lax.axis_index("c")

    pltpu.sync_copy(pos_hbm, pos_v)

    @pl.loop(0, seq_per_w)
    def _seq_loop(s):
        base = (wid * seq_per_w + s) * SEQ
        pltpu.sync_copy(idx_hbm.at[pl.ds(base, SEQ)], idx_v)
        copies = []
        for j in range(SEQ // GATHER_CHUNK):
            copies.append(
                pltpu.async_copy(
                    table_hbm.at[idx_v.at[pl.ds(j * GATHER_CHUNK, GATHER_CHUNK)]],
                    buf.at[pl.ds(j * GATHER_CHUNK, GATHER_CHUNK), :],
                    sem,
                )
            )
        for c in copies:
            c.wait()

        @pl.loop(0, SEQ)
        def _row_loop(r):
            for c in range(DIM // 16):
                sl = pl.ds(c * 16, 16)
                buf[r, sl] = buf[r, sl] + pos_v[r, sl]

        pltpu.sync_copy(buf, out_hbm.at[pl.ds(base, SEQ)])


def kernel(inputs, token_table, pos_table):
    batch, seq = inputs.shape
    flat_idx = inputs.reshape(batch * seq)
    mesh = plsc.VectorSubcoreMesh(
        core_axis_name="c",
        subcore_axis_name="s",
        num_cores=NUM_CORES,
        num_subcores=NUM_SUBCORES,
    )
    out = pl.kernel(
        _body,
        out_type=jax.ShapeDtypeStruct((batch * seq, DIM), jnp.float32),
        mesh=mesh,
        scratch_types=[
            pltpu.VMEM((SEQ,), jnp.int32),
            pltpu.VMEM((SEQ, DIM), jnp.float32),
            pltpu.VMEM((SEQ, DIM), jnp.float32),
            pltpu.SemaphoreType.DMA,
        ],
        compiler_params=pltpu.CompilerParams(use_tc_tiling_on_sc=False),
    )(token_table, flat_idx, pos_table)
    return out.reshape(batch, seq, DIM)


# speedup vs baseline: 4.1386x; 1.3335x over previous
import functools

import jax
import jax.numpy as jnp
from jax import lax
from jax.experimental import pallas as pl
from jax.experimental.pallas import tpu as pltpu
from jax.experimental.pallas import tpu_sc as plsc

SEQ = 200
DIM = 64
NUM_CORES = 2
NUM_SUBCORES = 16
NUM_WORKERS = NUM_CORES * NUM_SUBCORES
GATHER_CHUNK = 40
NCHUNK = SEQ // GATHER_CHUNK


def _body(table_hbm, idx_hbm, pos_hbm, out_hbm,
          idx_v, gbuf, sbuf, pos_v, gsems, ssems):
    n_seq = idx_hbm.shape[0] // SEQ
    seq_per_w = n_seq // NUM_WORKERS
    wid = lax.axis_index("s") * NUM_CORES + lax.axis_index("c")
    my_base = wid * seq_per_w * SEQ

    pltpu.sync_copy(pos_hbm, pos_v)
    pltpu.sync_copy(idx_hbm.at[pl.ds(my_base, seq_per_w * SEQ)], idx_v)

    def fire_gathers(s, b):
        for j in range(NCHUNK):
            pltpu.async_copy(
                table_hbm.at[idx_v.at[pl.ds(s * SEQ + j * GATHER_CHUNK,
                                            GATHER_CHUNK)]],
                gbuf.at[b, pl.ds(j * GATHER_CHUNK, GATHER_CHUNK), :],
                gsems.at[b],
            )

    def wait_gathers(b):
        for j in range(NCHUNK):
            pltpu.make_async_copy(
                table_hbm.at[idx_v.at[pl.ds(j * GATHER_CHUNK, GATHER_CHUNK)]],
                gbuf.at[b, pl.ds(j * GATHER_CHUNK, GATHER_CHUNK), :],
                gsems.at[b],
            ).wait()

    def wait_store(b):
        pltpu.make_async_copy(
            sbuf.at[b], out_hbm.at[pl.ds(0, SEQ)], ssems.at[b],
        ).wait()

    for b in range(2):
        fire_gathers(b, b)

    @pl.loop(0, seq_per_w // 2)
    def _pair_loop(i):
        for b in range(2):
            cur = i * 2 + b
            wait_gathers(b)

            @pl.when(i >= 1)
            def _():
                wait_store(b)

            @pl.loop(0, SEQ)
            def _row_loop(r):
                for c in range(DIM // 16):
                    sl = pl.ds(c * 16, 16)
                    sbuf[b, r, sl] = gbuf[b, r, sl] + pos_v[r, sl]

            @pl.when(i < seq_per_w // 2 - 1)
            def _():
                fire_gathers(cur + 2, b)

            pltpu.async_copy(
                sbuf.at[b],
                out_hbm.at[pl.ds(my_base + cur * SEQ, SEQ)],
                ssems.at[b],
            )

    for b in range(2):
        wait_store(b)


def kernel(inputs, token_table, pos_table):
    batch, seq = inputs.shape
    flat_idx = inputs.reshape(batch * seq)
    seq_per_w = batch // NUM_WORKERS
    mesh = plsc.VectorSubcoreMesh(
        core_axis_name="c",
        subcore_axis_name="s",
        num_cores=NUM_CORES,
        num_subcores=NUM_SUBCORES,
    )
    out = pl.kernel(
        _body,
        out_type=jax.ShapeDtypeStruct((batch * seq, DIM), jnp.float32),
        mesh=mesh,
        scratch_types=[
            pltpu.VMEM((seq_per_w * SEQ,), jnp.int32),
            pltpu.VMEM((2, SEQ, DIM), jnp.float32),
            pltpu.VMEM((2, SEQ, DIM), jnp.float32),
            pltpu.VMEM((SEQ, DIM), jnp.float32),
            pltpu.SemaphoreType.DMA((2,)),
            pltpu.SemaphoreType.DMA((2,)),
        ],
        compiler_params=pltpu.CompilerParams(use_tc_tiling_on_sc=False),
    )(token_table, flat_idx, pos_table)
    return out.reshape(batch, seq, DIM)
